# baseline (speedup 1.0000x reference)
import functools

import jax
import jax.numpy as jnp
from jax import lax
from jax.experimental import pallas as pl
from jax.experimental.pallas import tpu as pltpu
from jax.experimental.pallas import tpu_sc as plsc

NUM_FACTORS = 32
BATCH = 16384
NC = 2
NS = 16
NW = NC * NS
BPW = BATCH // NW
CHUNK = 128
NCHUNKS = BPW // CHUNK


def _bpr_body(uid_hbm, iid_hbm, jid_hbm, uf_hbm, itf_hbm,
              out_i_hbm, out_j_hbm,
              uid_v, iid_v, jid_v, urows_v, irows_v, jrows_v,
              oi_v, oj_v, sem):
    wid = lax.axis_index("s") * NC + lax.axis_index("c")
    base = wid * BPW

    pltpu.sync_copy(uid_hbm.at[pl.ds(base, BPW)], uid_v)
    pltpu.sync_copy(iid_hbm.at[pl.ds(base, BPW)], iid_v)
    pltpu.sync_copy(jid_hbm.at[pl.ds(base, BPW)], jid_v)

    copies = []
    for t in range(NCHUNKS):
        sl = pl.ds(t * CHUNK, CHUNK)
        copies.append(pltpu.async_copy(uf_hbm.at[uid_v.at[sl]],
                                       urows_v.at[sl, :], sem))
        copies.append(pltpu.async_copy(itf_hbm.at[iid_v.at[sl]],
                                       irows_v.at[sl, :], sem))
        copies.append(pltpu.async_copy(itf_hbm.at[jid_v.at[sl]],
                                       jrows_v.at[sl, :], sem))
    for c in copies:
        c.wait()

    lane = lax.iota(jnp.int32, 16)
    last = lane == 15

    def body(r, _):
        u0 = urows_v[r, pl.ds(0, 16)]
        u1 = urows_v[r, pl.ds(16, 16)]
        i0 = irows_v[r, pl.ds(0, 16)]
        i1 = irows_v[r, pl.ds(16, 16)]
        j0 = jrows_v[r, pl.ds(0, 16)]
        j1 = jrows_v[r, pl.ds(16, 16)]
        ci = plsc.cumsum(u0 * i0 + u1 * i1)
        cj = plsc.cumsum(u0 * j0 + u1 * j1)
        idx = jnp.full((16,), r, jnp.int32)
        plsc.store_scatter(oi_v, [idx], ci, mask=last)
        plsc.store_scatter(oj_v, [idx], cj, mask=last)
        return _

    lax.fori_loop(0, BPW, body, 0)

    pltpu.sync_copy(oi_v, out_i_hbm.at[pl.ds(base, BPW)])
    pltpu.sync_copy(oj_v, out_j_hbm.at[pl.ds(base, BPW)])


@jax.jit
def _bpr_sc(user_ids, item_ids_i, item_ids_j, user_factors, item_factors):
    run = pl.kernel(
        _bpr_body,
        out_type=(jax.ShapeDtypeStruct((BATCH,), jnp.float32),
                  jax.ShapeDtypeStruct((BATCH,), jnp.float32)),
        mesh=plsc.VectorSubcoreMesh(core_axis_name="c", subcore_axis_name="s"),
        compiler_params=pltpu.CompilerParams(needs_layout_passes=False,
                                             use_tc_tiling_on_sc=False),
        scratch_types=[
            pltpu.VMEM((BPW,), jnp.int32),
            pltpu.VMEM((BPW,), jnp.int32),
            pltpu.VMEM((BPW,), jnp.int32),
            pltpu.VMEM((BPW, NUM_FACTORS), jnp.float32),
            pltpu.VMEM((BPW, NUM_FACTORS), jnp.float32),
            pltpu.VMEM((BPW, NUM_FACTORS), jnp.float32),
            pltpu.VMEM((BPW,), jnp.float32),
            pltpu.VMEM((BPW,), jnp.float32),
            pltpu.SemaphoreType.DMA,
        ],
    )
    return run(user_ids, item_ids_i, item_ids_j, user_factors, item_factors)


def kernel(user_ids, item_ids_i, item_ids_j, user_factors, item_factors):
    return _bpr_sc(user_ids, item_ids_i, item_ids_j,
                   user_factors, item_factors)
